# baseline (speedup 1.0000x reference)
import functools

import jax
import jax.numpy as jnp
from jax import lax
from jax.experimental import pallas as pl
from jax.experimental.pallas import tpu as pltpu
from jax.experimental.pallas import tpu_sc as plsc

_NC = 2
_NS = 16
_NW = _NC * _NS
_L = 16


def _sc_mesh():
    return plsc.VectorSubcoreMesh(
        core_axis_name="c", subcore_axis_name="s",
        num_cores=_NC, num_subcores=_NS)


def _zero_ref(ref):
    zeros = jnp.zeros((_L,), jnp.float32)

    def body(i, carry):
        ref[pl.ds(i * _L, _L)] = zeros
        return carry

    lax.fori_loop(0, ref.shape[0] // _L, body, 0, unroll=4)


def _fast_rsqrt(x):
    i = plsc.bitcast(x, jnp.int32)
    i = jnp.int32(0x5F3759DF) - (i >> 1)
    y = plsc.bitcast(i, jnp.float32)
    for _ in range(3):
        y = y * (1.5 - 0.5 * x * y * y)
    return y


def _mega_body(src_hbm, dst_hbm, h_hbm,
               accp_hbm, g_hbm, dis_hbm,
               hist_v, src_v, dst_v, g_v, acc_v, dp_v, hseg_v, gseg_v,
               shared_deg, shared_g):
    eph = hist_v.shape[0]
    epw = src_v.shape[0]
    npad = g_v.shape[0]
    seg = npad // _NS
    cid = lax.axis_index("c")
    sid = lax.axis_index("s")
    wid = sid * _NC + cid

    pltpu.sync_copy(dst_hbm.at[pl.ds(sid * eph, eph)], hist_v)
    _zero_ref(acc_v)
    ones = jnp.ones((_L,), jnp.float32)

    def hist_body(i, carry):
        d = hist_v[pl.ds(i * _L, _L)]
        plsc.addupdate_scatter(acc_v, [d], ones)
        return carry

    lax.fori_loop(0, eph // _L, hist_body, 0, unroll=4)
    pltpu.sync_copy(acc_v, shared_deg.at[sid])
    pltpu.sync_copy(src_hbm.at[pl.ds(wid * epw, epw)], src_v)
    pltpu.sync_copy(dst_hbm.at[pl.ds(wid * epw, epw)], dst_v)
    pltpu.sync_copy(h_hbm.at[pl.ds(sid * seg, seg)], hseg_v)
    plsc.subcore_barrier()

    pltpu.sync_copy(shared_deg.at[:, pl.ds(sid * seg, seg)], dp_v)

    def seg_body(v, carry):
        tot = jnp.ones((_L,), jnp.float32)
        for r in range(_NS):
            tot = tot + dp_v[r, pl.ds(v * _L, _L)]
        dis = _fast_rsqrt(tot)
        gseg_v[pl.ds(v * _L, _L)] = dis * hseg_v[pl.ds(v * _L, _L)]
        hseg_v[pl.ds(v * _L, _L)] = dis
        return carry

    lax.fori_loop(0, seg // _L, seg_body, 0, unroll=2)
    pltpu.sync_copy(gseg_v, shared_g.at[pl.ds(sid * seg, seg)])

    @pl.when(cid == 0)
    def _():
        pltpu.sync_copy(gseg_v, g_hbm.at[pl.ds(sid * seg, seg)])
        pltpu.sync_copy(hseg_v, dis_hbm.at[pl.ds(sid * seg, seg)])

    plsc.subcore_barrier()
    pltpu.sync_copy(shared_g, g_v)

    _zero_ref(acc_v)

    def msg_body(i, carry):
        s = src_v[pl.ds(i * _L, _L)]
        d = dst_v[pl.ds(i * _L, _L)]
        vals = plsc.load_gather(g_v, [s])
        plsc.addupdate_scatter(acc_v, [d], vals)
        return carry

    lax.fori_loop(0, epw // _L, msg_body, 0, unroll=4)
    pltpu.sync_copy(acc_v, accp_hbm.at[wid])


def _mega_call(src, dst, h_pad, npad):
    e = src.shape[0]
    eph = e // _NS
    epw = e // _NW
    seg = npad // _NS
    fn = pl.kernel(
        _mega_body,
        out_type=(
            jax.ShapeDtypeStruct((_NW, npad), jnp.float32),
            jax.ShapeDtypeStruct((npad,), jnp.float32),
            jax.ShapeDtypeStruct((npad,), jnp.float32),
        ),
        mesh=_sc_mesh(),
        compiler_params=pltpu.CompilerParams(needs_layout_passes=False),
        scratch_types=[
            pltpu.VMEM((eph,), jnp.int32),
            pltpu.VMEM((epw,), jnp.int32),
            pltpu.VMEM((epw,), jnp.int32),
            pltpu.VMEM((npad,), jnp.float32),
            pltpu.VMEM((npad,), jnp.float32),
            pltpu.VMEM((_NS, seg), jnp.float32),
            pltpu.VMEM((seg,), jnp.float32),
            pltpu.VMEM((seg,), jnp.float32),
            pltpu.VMEM_SHARED((_NS, npad), jnp.float32),
            pltpu.VMEM_SHARED((npad,), jnp.float32),
        ],
    )
    return fn(src, dst, h_pad)


def _mv_body(x_ref, w_ref, h_ref):
    h_ref[...] = lax.dot_general(
        w_ref[...], x_ref[...], (((1,), (1,)), ((), ())),
        preferred_element_type=jnp.float32)


def _mv_call(x, w_row):
    n = x.shape[0]
    return pl.pallas_call(
        _mv_body,
        out_shape=jax.ShapeDtypeStruct((1, n), jnp.float32),
    )(x, w_row)


def _fin_body(accp_ref, g_ref, dis_ref, b_ref, out_ref):
    n = out_ref.shape[1]
    tot = jnp.sum(accp_ref[:, :n], axis=0, keepdims=True)
    z = dis_ref[:, :n] * (tot + g_ref[:, :n]) + b_ref[0, 0]
    out_ref[...] = jnp.maximum(z, 0.0) + jnp.log1p(jnp.exp(-jnp.abs(z)))


def _fin_call(accp, g_row, dis_row, b, n):
    return pl.pallas_call(
        _fin_body,
        out_shape=jax.ShapeDtypeStruct((1, n), jnp.float32),
    )(accp, g_row, dis_row, b.reshape(1, 1))


@jax.jit
def kernel(x, edge_index, W, b):
    n, d = x.shape
    npad = -(-n // (_NS * _L)) * (_NS * _L)
    src = edge_index[0]
    dst = edge_index[1]
    h_row = _mv_call(x, W.reshape(1, d))
    h_pad = jnp.pad(h_row.reshape(n), (0, npad - n))
    accp, g, dis = _mega_call(src, dst, h_pad, npad)
    out_row = _fin_call(accp, g.reshape(1, npad), dis.reshape(1, npad), b, n)
    return out_row.reshape(n, 1)


# speedup vs baseline: 1.3081x; 1.3081x over previous
import functools

import jax
import jax.numpy as jnp
from jax import lax
from jax.experimental import pallas as pl
from jax.experimental.pallas import tpu as pltpu
from jax.experimental.pallas import tpu_sc as plsc

_NC = 2
_NS = 16
_NW = _NC * _NS
_L = 16


def _sc_mesh():
    return plsc.VectorSubcoreMesh(
        core_axis_name="c", subcore_axis_name="s",
        num_cores=_NC, num_subcores=_NS)


def _wid():
    return lax.axis_index("s") * _NC + lax.axis_index("c")


def _zero_ref(ref):
    zeros = jnp.zeros((_L,), jnp.float32)

    def body(i, carry):
        ref[pl.ds(i * _L, _L)] = zeros
        return carry

    lax.fori_loop(0, ref.shape[0] // _L, body, 0, unroll=4)


def _deg_body(ei_hbm, out_hbm, dst_v, acc_v):
    epw = dst_v.shape[0]
    e = ei_hbm.shape[0] // 2
    wid = _wid()
    pltpu.sync_copy(ei_hbm.at[pl.ds(e + wid * epw, epw)], dst_v)
    _zero_ref(acc_v)
    ones = jnp.ones((_L,), jnp.float32)

    def body(i, carry):
        d = dst_v[pl.ds(i * _L, _L)]
        plsc.addupdate_scatter(acc_v, [d], ones)
        return carry

    lax.fori_loop(0, epw // _L, body, 0, unroll=4)
    pltpu.sync_copy(acc_v, out_hbm.at[wid])


def _deg_call(ei_flat, npad):
    e = ei_flat.shape[0] // 2
    epw = e // _NW
    fn = pl.kernel(
        _deg_body,
        out_type=jax.ShapeDtypeStruct((_NW, npad), jnp.float32),
        mesh=_sc_mesh(),
        compiler_params=pltpu.CompilerParams(needs_layout_passes=False),
        scratch_types=[
            pltpu.VMEM((epw,), jnp.int32),
            pltpu.VMEM((npad,), jnp.float32),
        ],
    )
    return fn(ei_flat)


def _msg_body(ei_hbm, g_hbm, out_hbm, src_v, dst_v, g_v, acc_v):
    epw = src_v.shape[0]
    e = ei_hbm.shape[0] // 2
    wid = _wid()
    pltpu.sync_copy(g_hbm, g_v)
    pltpu.sync_copy(ei_hbm.at[pl.ds(wid * epw, epw)], src_v)
    pltpu.sync_copy(ei_hbm.at[pl.ds(e + wid * epw, epw)], dst_v)
    _zero_ref(acc_v)

    def body(i, carry):
        s = src_v[pl.ds(i * _L, _L)]
        d = dst_v[pl.ds(i * _L, _L)]
        vals = plsc.load_gather(g_v, [s])
        plsc.addupdate_scatter(acc_v, [d], vals)
        return carry

    lax.fori_loop(0, epw // _L, body, 0, unroll=4)
    pltpu.sync_copy(acc_v, out_hbm.at[wid])


def _msg_call(ei_flat, g, npad):
    e = ei_flat.shape[0] // 2
    epw = e // _NW
    fn = pl.kernel(
        _msg_body,
        out_type=jax.ShapeDtypeStruct((_NW, npad), jnp.float32),
        mesh=_sc_mesh(),
        compiler_params=pltpu.CompilerParams(needs_layout_passes=False),
        scratch_types=[
            pltpu.VMEM((epw,), jnp.int32),
            pltpu.VMEM((epw,), jnp.int32),
            pltpu.VMEM((npad,), jnp.float32),
            pltpu.VMEM((npad,), jnp.float32),
        ],
    )
    return fn(ei_flat, g)


def _prep_body(x_ref, w_ref, degp_ref, g_ref, dis_ref):
    n = x_ref.shape[0]
    deg = jnp.sum(degp_ref[...], axis=0, keepdims=True) + 1.0
    dis = lax.rsqrt(deg)
    h = lax.dot_general(w_ref[...], x_ref[...], (((1,), (1,)), ((), ())),
                        preferred_element_type=jnp.float32)
    g_ref[...] = jnp.zeros_like(g_ref)
    g_ref[:, :n] = dis[:, :n] * h
    dis_ref[...] = dis


def _prep_call(x, w_row, degp, npad):
    shape = jax.ShapeDtypeStruct((1, npad), jnp.float32)
    return pl.pallas_call(
        _prep_body,
        out_shape=(shape, shape),
    )(x, w_row, degp)


def _fin_body(accp_ref, g_ref, dis_ref, b_ref, out_ref):
    n = out_ref.shape[1]
    tot = jnp.sum(accp_ref[:, :n], axis=0, keepdims=True)
    z = dis_ref[:, :n] * (tot + g_ref[:, :n]) + b_ref[0, 0]
    out_ref[...] = jnp.maximum(z, 0.0) + jnp.log1p(jnp.exp(-jnp.abs(z)))


def _fin_call(accp, g_row, dis_row, b, n):
    return pl.pallas_call(
        _fin_body,
        out_shape=jax.ShapeDtypeStruct((1, n), jnp.float32),
    )(accp, g_row, dis_row, b.reshape(1, 1))


@jax.jit
def kernel(x, edge_index, W, b):
    n, d = x.shape
    e = edge_index.shape[1]
    npad = -(-n // (_NS * _L)) * (_NS * _L)
    ei_flat = edge_index.reshape(2 * e)
    degp = _deg_call(ei_flat, npad)
    g_row, dis_row = _prep_call(x, W.reshape(1, d), degp, npad)
    accp = _msg_call(ei_flat, g_row.reshape(npad), npad)
    out_row = _fin_call(accp, g_row, dis_row, b, n)
    return out_row.reshape(n, 1)


# speedup vs baseline: 1.3887x; 1.0616x over previous
import functools

import jax
import jax.numpy as jnp
from jax import lax
from jax.experimental import pallas as pl
from jax.experimental.pallas import tpu as pltpu
from jax.experimental.pallas import tpu_sc as plsc

_NC = 2
_NS = 16
_NW = _NC * _NS
_L = 16


def _sc_mesh():
    return plsc.VectorSubcoreMesh(
        core_axis_name="c", subcore_axis_name="s",
        num_cores=_NC, num_subcores=_NS)


def _wid():
    return lax.axis_index("s") * _NC + lax.axis_index("c")


def _zero_ref(ref):
    zeros = jnp.zeros((_L,), jnp.float32)

    def body(i, carry):
        ref[pl.ds(i * _L, _L)] = zeros
        return carry

    lax.fori_loop(0, ref.shape[0] // _L, body, 0, unroll=4)


_ECH = 128


def _chunk_range(ei_hbm):
    ncw = ei_hbm.shape[1] // _ECH
    wid = _wid()
    c0 = wid * ncw // _NW
    c1 = (wid + 1) * ncw // _NW
    return c0, c1


def _deg_body(ei_hbm, out_hbm, ed_v, acc_v):
    c0, c1 = _chunk_range(ei_hbm)
    pltpu.sync_copy(ei_hbm.at[:, pl.ds(c0 * _ECH, ed_v.shape[1])], ed_v)
    _zero_ref(acc_v)
    ones = jnp.ones((_L,), jnp.float32)

    def body(ch, carry):
        for v in range(_ECH // _L):
            d = ed_v[1, pl.ds(ch * _ECH + v * _L, _L)]
            plsc.addupdate_scatter(acc_v, [d], ones)
        return carry

    lax.fori_loop(0, c1 - c0, body, 0)
    pltpu.sync_copy(acc_v, out_hbm.at[_wid()])


def _deg_call(edge_index, npad):
    e = edge_index.shape[1]
    cmax = e // _ECH // _NW + 1
    fn = pl.kernel(
        _deg_body,
        out_type=jax.ShapeDtypeStruct((_NW, npad), jnp.float32),
        mesh=_sc_mesh(),
        compiler_params=pltpu.CompilerParams(needs_layout_passes=False),
        scratch_types=[
            pltpu.VMEM((2, cmax * _ECH), jnp.int32),
            pltpu.VMEM((npad,), jnp.float32),
        ],
    )
    return fn(edge_index)


def _msg_body(ei_hbm, g_hbm, out_hbm, ed_v, g_v, acc_v):
    c0, c1 = _chunk_range(ei_hbm)
    pltpu.sync_copy(g_hbm, g_v)
    pltpu.sync_copy(ei_hbm.at[:, pl.ds(c0 * _ECH, ed_v.shape[1])], ed_v)
    _zero_ref(acc_v)

    def body(ch, carry):
        for v in range(_ECH // _L):
            s = ed_v[0, pl.ds(ch * _ECH + v * _L, _L)]
            d = ed_v[1, pl.ds(ch * _ECH + v * _L, _L)]
            vals = plsc.load_gather(g_v, [s])
            plsc.addupdate_scatter(acc_v, [d], vals)
        return carry

    lax.fori_loop(0, c1 - c0, body, 0)
    pltpu.sync_copy(acc_v, out_hbm.at[_wid()])


def _msg_call(edge_index, g, npad):
    e = edge_index.shape[1]
    cmax = e // _ECH // _NW + 1
    fn = pl.kernel(
        _msg_body,
        out_type=jax.ShapeDtypeStruct((_NW, npad), jnp.float32),
        mesh=_sc_mesh(),
        compiler_params=pltpu.CompilerParams(needs_layout_passes=False),
        scratch_types=[
            pltpu.VMEM((2, cmax * _ECH), jnp.int32),
            pltpu.VMEM((npad,), jnp.float32),
            pltpu.VMEM((npad,), jnp.float32),
        ],
    )
    return fn(edge_index, g)


def _prep_body(x_ref, w_ref, degp_ref, g_ref, dis_ref):
    n = x_ref.shape[0]
    deg = jnp.sum(degp_ref[...], axis=0, keepdims=True) + 1.0
    dis = lax.rsqrt(deg)
    h = lax.dot_general(w_ref[...], x_ref[...], (((1,), (1,)), ((), ())),
                        preferred_element_type=jnp.float32)
    g_ref[...] = jnp.zeros_like(g_ref)
    g_ref[:, :n] = dis[:, :n] * h
    dis_ref[...] = dis


def _prep_call(x, w_row, degp, npad):
    shape = jax.ShapeDtypeStruct((1, npad), jnp.float32)
    return pl.pallas_call(
        _prep_body,
        out_shape=(shape, shape),
    )(x, w_row, degp)


def _fin_body(accp_ref, g_ref, dis_ref, b_ref, out_ref):
    n = out_ref.shape[1]
    tot = jnp.sum(accp_ref[:, :n], axis=0, keepdims=True)
    z = dis_ref[:, :n] * (tot + g_ref[:, :n]) + b_ref[0, 0]
    out_ref[...] = jnp.maximum(z, 0.0) + jnp.log1p(jnp.exp(-jnp.abs(z)))


def _fin_call(accp, g_row, dis_row, b, n):
    return pl.pallas_call(
        _fin_body,
        out_shape=jax.ShapeDtypeStruct((1, n), jnp.float32),
    )(accp, g_row, dis_row, b.reshape(1, 1))


@jax.jit
def kernel(x, edge_index, W, b):
    n, d = x.shape
    e = edge_index.shape[1]
    npad = -(-n // (_NS * _L)) * (_NS * _L)
    degp = _deg_call(edge_index, npad)
    g_row, dis_row = _prep_call(x, W.reshape(1, d), degp, npad)
    accp = _msg_call(edge_index, g_row.reshape(npad), npad)
    out_row = _fin_call(accp, g_row, dis_row, b, n)
    return out_row.reshape(n, 1)
